# baseline (speedup 1.0000x reference)
import jax
import jax.numpy as jnp
from jax import lax
from jax.experimental import pallas as pl
from jax.experimental.pallas import tpu as pltpu
from jax.experimental.pallas import tpu_sc as plsc

NC = 2
NS = 16
L = 16
NW = NC * NS
B = 16384
D = 128
BPW = B // NW
CHUNK = 64
NCHUNK = BPW // CHUNK
EPS = 1e-6
GAMMA = 1.0


def _rsqrt(x):
    i = lax.bitcast_convert_type(x, jnp.int32)
    i = jnp.int32(0x5F3759DF) - lax.shift_right_logical(i, 1)
    y = lax.bitcast_convert_type(i, jnp.float32)
    for _ in range(2):
        y = y * (1.5 - 0.5 * x * y * y)
    return y


def _tec_body(head_r, rel_r, tail_r, nh_r, etab, rtab, out_r,
              idx_h, idx_r, idx_t, idx_n, rh, rr, rt, rn,
              rh2, rr2, rt2, rn2, rh3, rr3, rt3, rn3, loss_buf,
              sem, sem2, sem3):
    cid = lax.axis_index("c")
    sid = lax.axis_index("s")
    wid = sid * NC + cid
    base = wid * BPW

    icps = [
        pltpu.async_copy(head_r.at[pl.ds(base, BPW)], idx_h, sem),
        pltpu.async_copy(rel_r.at[pl.ds(base, BPW)], idx_r, sem2),
        pltpu.async_copy(tail_r.at[pl.ds(base, BPW)], idx_t, sem),
        pltpu.async_copy(nh_r.at[pl.ds(base, BPW)], idx_n, sem2),
    ]
    for cp in icps:
        cp.wait()

    bufs = ((rh, rr, rt, rn), (rh2, rr2, rt2, rn2), (rh3, rr3, rt3, rn3))
    sems = (sem, sem2, sem3)

    def start(ck):
        o = ck * CHUNK
        bh, br, bt, bn = bufs[ck % 3]
        s = sems[ck % 3]
        return [
            pltpu.async_copy(etab.at[idx_h.at[pl.ds(o, CHUNK)]], bh, s),
            pltpu.async_copy(rtab.at[idx_r.at[pl.ds(o, CHUNK)]], br, s),
            pltpu.async_copy(etab.at[idx_t.at[pl.ds(o, CHUNK)]], bt, s),
            pltpu.async_copy(etab.at[idx_n.at[pl.ds(o, CHUNK)]], bn, s),
        ]

    loss = jnp.zeros((L,), jnp.float32)
    zero = jnp.zeros((L,), jnp.float32)
    iota = lax.broadcasted_iota(jnp.int32, (L,), 0)

    inflight = [start(0), start(1)]
    for ck in range(NCHUNK):
        nxt = start(ck + 2) if ck + 2 < NCHUNK else []
        for cp in inflight.pop(0):
            cp.wait()
        inflight.append(nxt)
        rhc, rrc, rtc, rnc = bufs[ck % 3]

        def eg_body(eg, loss):
            row = eg * L + iota

            @plsc.parallel_loop(0, D, unroll=2, carry=(zero,) * 9)
            def dots(d, c):
                hr, ht, rt_, nr, nt, sh, sr, st, sn = c
                col = (iota + d) & (D - 1)
                h = plsc.load_gather(rhc, [row, col])
                r = plsc.load_gather(rrc, [row, col])
                t = plsc.load_gather(rtc, [row, col])
                n = plsc.load_gather(rnc, [row, col])
                return (hr + h * r, ht + h * t, rt_ + r * t,
                        nr + n * r, nt + n * t,
                        sh + h * h, sr + r * r, st + t * t, sn + n * n)

            hr, ht, rt_, nr, nt, sh, sr, st, sn = dots
            ah = _rsqrt(jnp.maximum(sh, 1e-24))
            ar = _rsqrt(jnp.maximum(sr, 1e-24))
            at = _rsqrt(jnp.maximum(st, 1e-24))
            an = _rsqrt(jnp.maximum(sn, 1e-24))

            rtbc = rt_ * (ar * at)
            d1sq = (3.0 + D * EPS * EPS
                    + 2.0 * ((hr * ah) * ar - (ht * ah) * at - rtbc))
            d2sq = (3.0 + D * EPS * EPS
                    + 2.0 * ((nr * an) * ar - (nt * an) * at - rtbc))
            s1 = jnp.maximum(d1sq, 1e-30)
            s2 = jnp.maximum(d2sq, 1e-30)
            d1 = s1 * _rsqrt(s1)
            d2 = s2 * _rsqrt(s2)
            return loss + jnp.maximum(GAMMA + d1 - d2, 0.0)

        loss = lax.fori_loop(0, CHUNK // L, eg_body, loss)

    loss_buf[...] = loss
    pltpu.sync_copy(loss_buf, out_r.at[wid])


@jax.jit
def _transe_loss_partials(head, rel, tail, negative_head, entity_table,
                          relation_table):
    mesh = plsc.VectorSubcoreMesh(
        core_axis_name="c", subcore_axis_name="s", num_cores=NC,
        num_subcores=NS)
    f = pl.kernel(
        _tec_body,
        out_type=jax.ShapeDtypeStruct((NW, L), jnp.float32),
        mesh=mesh,
        compiler_params=pltpu.CompilerParams(needs_layout_passes=False),
        scratch_types=[
            pltpu.VMEM((BPW,), jnp.int32),
            pltpu.VMEM((BPW,), jnp.int32),
            pltpu.VMEM((BPW,), jnp.int32),
            pltpu.VMEM((BPW,), jnp.int32),
            pltpu.VMEM((CHUNK, D), jnp.float32),
            pltpu.VMEM((CHUNK, D), jnp.float32),
            pltpu.VMEM((CHUNK, D), jnp.float32),
            pltpu.VMEM((CHUNK, D), jnp.float32),
            pltpu.VMEM((CHUNK, D), jnp.float32),
            pltpu.VMEM((CHUNK, D), jnp.float32),
            pltpu.VMEM((CHUNK, D), jnp.float32),
            pltpu.VMEM((CHUNK, D), jnp.float32),
            pltpu.VMEM((CHUNK, D), jnp.float32),
            pltpu.VMEM((CHUNK, D), jnp.float32),
            pltpu.VMEM((CHUNK, D), jnp.float32),
            pltpu.VMEM((CHUNK, D), jnp.float32),
            pltpu.VMEM((L,), jnp.float32),
            pltpu.SemaphoreType.DMA,
            pltpu.SemaphoreType.DMA,
            pltpu.SemaphoreType.DMA,
        ],
    )
    return f(head, rel, tail, negative_head, entity_table, relation_table)


def kernel(head, rel, tail, negative_head, negative_tail, entity_table,
           relation_table):
    del negative_tail
    partials = _transe_loss_partials(head, rel, tail, negative_head,
                                     entity_table, relation_table)
    return jnp.sum(partials)


# speedup vs baseline: 1.0137x; 1.0137x over previous
import jax
import jax.numpy as jnp
from jax import lax
from jax.experimental import pallas as pl
from jax.experimental.pallas import tpu as pltpu
from jax.experimental.pallas import tpu_sc as plsc

NC = 2
NS = 16
L = 16
NW = NC * NS
B = 16384
D = 128
BPW = B // NW
CHUNK = 64
NCHUNK = BPW // CHUNK
EPS = 1e-6
GAMMA = 1.0


def _rsqrt(x):
    i = lax.bitcast_convert_type(x, jnp.int32)
    i = jnp.int32(0x5F3759DF) - lax.shift_right_logical(i, 1)
    y = lax.bitcast_convert_type(i, jnp.float32)
    for _ in range(2):
        y = y * (1.5 - 0.5 * x * y * y)
    return y


def _tec_body(ht_r, rel_r, nh_r, etab, rtab, out_r,
              idx_ht, idx_r, idx_n, rht, rr, rn,
              rht2, rr2, rn2, loss_buf, sem, sem2):
    cid = lax.axis_index("c")
    sid = lax.axis_index("s")
    wid = sid * NC + cid
    base = wid * BPW

    icps = [
        pltpu.async_copy(ht_r.at[pl.ds(wid * NCHUNK, NCHUNK)], idx_ht, sem),
        pltpu.async_copy(rel_r.at[pl.ds(base, BPW)], idx_r, sem2),
        pltpu.async_copy(nh_r.at[pl.ds(base, BPW)], idx_n, sem2),
    ]
    for cp in icps:
        cp.wait()

    bufs = ((rht, rr, rn), (rht2, rr2, rn2))
    sems = (sem, sem2)

    def start(ck):
        o = ck * CHUNK
        bht, br, bn = bufs[ck % 2]
        s = sems[ck % 2]
        return [
            pltpu.async_copy(etab.at[idx_ht.at[ck]], bht, s),
            pltpu.async_copy(rtab.at[idx_r.at[pl.ds(o, CHUNK)]], br, s),
            pltpu.async_copy(etab.at[idx_n.at[pl.ds(o, CHUNK)]], bn, s),
        ]

    loss = jnp.zeros((L,), jnp.float32)
    zero = jnp.zeros((L,), jnp.float32)
    iota = lax.broadcasted_iota(jnp.int32, (L,), 0)

    inflight = start(0)
    for ck in range(NCHUNK):
        nxt = start(ck + 1) if ck + 1 < NCHUNK else []
        for cp in inflight:
            cp.wait()
        inflight = nxt
        rhtc, rrc, rnc = bufs[ck % 2]

        def eg_body(eg, loss):
            row = eg * L + iota
            rowt = row + CHUNK

            @plsc.parallel_loop(0, D, unroll=2, carry=(zero,) * 9)
            def dots(d, c):
                hr, ht, rt_, nr, nt, sh, sr, st, sn = c
                col = (iota + d) & (D - 1)
                h = plsc.load_gather(rhtc, [row, col])
                r = plsc.load_gather(rrc, [row, col])
                t = plsc.load_gather(rhtc, [rowt, col])
                n = plsc.load_gather(rnc, [row, col])
                return (hr + h * r, ht + h * t, rt_ + r * t,
                        nr + n * r, nt + n * t,
                        sh + h * h, sr + r * r, st + t * t, sn + n * n)

            hr, ht, rt_, nr, nt, sh, sr, st, sn = dots
            ah = _rsqrt(jnp.maximum(sh, 1e-24))
            ar = _rsqrt(jnp.maximum(sr, 1e-24))
            at = _rsqrt(jnp.maximum(st, 1e-24))
            an = _rsqrt(jnp.maximum(sn, 1e-24))

            rtbc = rt_ * (ar * at)
            d1sq = (3.0 + D * EPS * EPS
                    + 2.0 * ((hr * ah) * ar - (ht * ah) * at - rtbc))
            d2sq = (3.0 + D * EPS * EPS
                    + 2.0 * ((nr * an) * ar - (nt * an) * at - rtbc))
            s1 = jnp.maximum(d1sq, 1e-30)
            s2 = jnp.maximum(d2sq, 1e-30)
            d1 = s1 * _rsqrt(s1)
            d2 = s2 * _rsqrt(s2)
            return loss + jnp.maximum(GAMMA + d1 - d2, 0.0)

        loss = lax.fori_loop(0, CHUNK // L, eg_body, loss)

    loss_buf[...] = loss
    pltpu.sync_copy(loss_buf, out_r.at[wid])


@jax.jit
def _transe_loss_partials(head_tail, rel, negative_head, entity_table,
                          relation_table):
    mesh = plsc.VectorSubcoreMesh(
        core_axis_name="c", subcore_axis_name="s", num_cores=NC,
        num_subcores=NS)
    f = pl.kernel(
        _tec_body,
        out_type=jax.ShapeDtypeStruct((NW, L), jnp.float32),
        mesh=mesh,
        compiler_params=pltpu.CompilerParams(needs_layout_passes=False),
        scratch_types=[
            pltpu.VMEM((NCHUNK, 2 * CHUNK), jnp.int32),
            pltpu.VMEM((BPW,), jnp.int32),
            pltpu.VMEM((BPW,), jnp.int32),
            pltpu.VMEM((2 * CHUNK, D), jnp.float32),
            pltpu.VMEM((CHUNK, D), jnp.float32),
            pltpu.VMEM((CHUNK, D), jnp.float32),
            pltpu.VMEM((2 * CHUNK, D), jnp.float32),
            pltpu.VMEM((CHUNK, D), jnp.float32),
            pltpu.VMEM((CHUNK, D), jnp.float32),
            pltpu.VMEM((L,), jnp.float32),
            pltpu.SemaphoreType.DMA,
            pltpu.SemaphoreType.DMA,
        ],
    )
    return f(head_tail, rel, negative_head, entity_table, relation_table)


def kernel(head, rel, tail, negative_head, negative_tail, entity_table,
           relation_table):
    del negative_tail
    head_tail = jnp.concatenate(
        [head.reshape(NW * NCHUNK, CHUNK), tail.reshape(NW * NCHUNK, CHUNK)],
        axis=1)
    partials = _transe_loss_partials(head_tail, rel, negative_head,
                                     entity_table, relation_table)
    return jnp.sum(partials)


# speedup vs baseline: 1.0314x; 1.0174x over previous
import jax
import jax.numpy as jnp
from jax import lax
from jax.experimental import pallas as pl
from jax.experimental.pallas import tpu as pltpu
from jax.experimental.pallas import tpu_sc as plsc

NC = 2
NS = 16
L = 16
NW = NC * NS
B = 16384
D = 128
BPW = B // NW
CHUNK = 64
NCHUNK = BPW // CHUNK
EPS = 1e-6
GAMMA = 1.0


def _rsqrt(x):
    i = lax.bitcast_convert_type(x, jnp.int32)
    i = jnp.int32(0x5F3759DF) - lax.shift_right_logical(i, 1)
    y = lax.bitcast_convert_type(i, jnp.float32)
    for _ in range(2):
        y = y * (1.5 - 0.5 * x * y * y)
    return y


def _tec_body(head_r, rel_r, tail_r, nh_r, etab, rtab, out_r,
              idx_h, idx_r, idx_t, idx_n, rh, rr, rt, rn,
              rh2, rr2, rt2, rn2, loss_buf, sem, sem2):
    cid = lax.axis_index("c")
    sid = lax.axis_index("s")
    wid = sid * NC + cid
    base = wid * BPW

    icps = [
        pltpu.async_copy(head_r.at[pl.ds(base, BPW)], idx_h, sem),
        pltpu.async_copy(rel_r.at[pl.ds(base, BPW)], idx_r, sem2),
        pltpu.async_copy(tail_r.at[pl.ds(base, BPW)], idx_t, sem),
        pltpu.async_copy(nh_r.at[pl.ds(base, BPW)], idx_n, sem2),
    ]
    for cp in icps:
        cp.wait()

    bufs = ((rh, rr, rt, rn), (rh2, rr2, rt2, rn2))
    sems = (sem, sem2)

    def start(ck):
        o = ck * CHUNK
        bh, br, bt, bn = bufs[ck % 2]
        s = sems[ck % 2]
        return [
            pltpu.async_copy(etab.at[idx_h.at[pl.ds(o, CHUNK)]], bh, s),
            pltpu.async_copy(rtab.at[idx_r.at[pl.ds(o, CHUNK)]], br, s),
            pltpu.async_copy(etab.at[idx_t.at[pl.ds(o, CHUNK)]], bt, s),
            pltpu.async_copy(etab.at[idx_n.at[pl.ds(o, CHUNK)]], bn, s),
        ]

    loss = jnp.zeros((L,), jnp.float32)
    zero = jnp.zeros((L,), jnp.float32)
    iota = lax.broadcasted_iota(jnp.int32, (L,), 0)

    inflight = start(0)
    for ck in range(NCHUNK):
        nxt = start(ck + 1) if ck + 1 < NCHUNK else []
        for cp in inflight:
            cp.wait()
        inflight = nxt
        rhc, rrc, rtc, rnc = bufs[ck % 2]

        def eg_body(eg, loss):
            row = eg * L + iota

            @plsc.parallel_loop(0, D, unroll=2, carry=(zero,) * 9)
            def dots(d, c):
                hr, ht, rt_, nr, nt, sh, sr, st, sn = c
                col = (iota + d) & (D - 1)
                h = plsc.load_gather(rhc, [row, col])
                r = plsc.load_gather(rrc, [row, col])
                t = plsc.load_gather(rtc, [row, col])
                n = plsc.load_gather(rnc, [row, col])
                return (hr + h * r, ht + h * t, rt_ + r * t,
                        nr + n * r, nt + n * t,
                        sh + h * h, sr + r * r, st + t * t, sn + n * n)

            hr, ht, rt_, nr, nt, sh, sr, st, sn = dots
            ah = _rsqrt(jnp.maximum(sh, 1e-24))
            ar = _rsqrt(jnp.maximum(sr, 1e-24))
            at = _rsqrt(jnp.maximum(st, 1e-24))
            an = _rsqrt(jnp.maximum(sn, 1e-24))

            rtbc = rt_ * (ar * at)
            d1sq = (3.0 + D * EPS * EPS
                    + 2.0 * ((hr * ah) * ar - (ht * ah) * at - rtbc))
            d2sq = (3.0 + D * EPS * EPS
                    + 2.0 * ((nr * an) * ar - (nt * an) * at - rtbc))
            s1 = jnp.maximum(d1sq, 1e-30)
            s2 = jnp.maximum(d2sq, 1e-30)
            d1 = s1 * _rsqrt(s1)
            d2 = s2 * _rsqrt(s2)
            return loss + jnp.maximum(GAMMA + d1 - d2, 0.0)

        loss = lax.fori_loop(0, CHUNK // L, eg_body, loss)

    loss_buf[...] = loss
    pltpu.sync_copy(loss_buf, out_r.at[wid])


@jax.jit
def _transe_loss_partials(head, rel, tail, negative_head, entity_table,
                          relation_table):
    mesh = plsc.VectorSubcoreMesh(
        core_axis_name="c", subcore_axis_name="s", num_cores=NC,
        num_subcores=NS)
    f = pl.kernel(
        _tec_body,
        out_type=jax.ShapeDtypeStruct((NW, L), jnp.float32),
        mesh=mesh,
        compiler_params=pltpu.CompilerParams(needs_layout_passes=False),
        scratch_types=[
            pltpu.VMEM((BPW,), jnp.int32),
            pltpu.VMEM((BPW,), jnp.int32),
            pltpu.VMEM((BPW,), jnp.int32),
            pltpu.VMEM((BPW,), jnp.int32),
            pltpu.VMEM((CHUNK, D), jnp.float32),
            pltpu.VMEM((CHUNK, D), jnp.float32),
            pltpu.VMEM((CHUNK, D), jnp.float32),
            pltpu.VMEM((CHUNK, D), jnp.float32),
            pltpu.VMEM((CHUNK, D), jnp.float32),
            pltpu.VMEM((CHUNK, D), jnp.float32),
            pltpu.VMEM((CHUNK, D), jnp.float32),
            pltpu.VMEM((CHUNK, D), jnp.float32),
            pltpu.VMEM((L,), jnp.float32),
            pltpu.SemaphoreType.DMA,
            pltpu.SemaphoreType.DMA,
        ],
    )
    return f(head, rel, tail, negative_head, entity_table, relation_table)


def kernel(head, rel, tail, negative_head, negative_tail, entity_table,
           relation_table):
    del negative_tail
    partials = _transe_loss_partials(head, rel, tail, negative_head,
                                     entity_table, relation_table)
    return jnp.sum(partials)
